# baseline (speedup 1.0000x reference)
import jax
import jax.numpy as jnp
from jax import lax
from jax.experimental import pallas as pl
from jax.experimental.pallas import tpu as pltpu
from jax.experimental.pallas import tpu_sc as plsc

T = 2048
D = 768
E = 64
K = 2
F = 512

TM = 128
NX = (T * K) // TM
N_TILES = NX + (E - 1)
NG = T * K

NW = 32
GB = (T * K) // NW
CB = T // NW
LANES = 16
DC = D // LANES


BS = 256
NB = T // BS
NT_PAD = 128


def _router_kernel(hs_ref, wg_ref, w_ref, slots_ref, sa_ref, sb_ref,
                   te_ref, tx_ref, ty_ref, sf_ref, st_ref, gs_ref, ge_ref):
    logits = jnp.dot(hs_ref[...], wg_ref[...], preferred_element_type=jnp.float32)
    m = jnp.max(logits, axis=1, keepdims=True)
    p = jnp.exp(logits - m)
    p = p / jnp.sum(p, axis=1, keepdims=True)
    iota = jax.lax.broadcasted_iota(jnp.int32, (T, E), 1)
    m1 = jnp.max(p, axis=1, keepdims=True)
    i1 = jnp.min(jnp.where(p == m1, iota, E), axis=1, keepdims=True)
    p2 = jnp.where(iota == i1, -1e30, p)
    m2 = jnp.max(p2, axis=1, keepdims=True)
    i2 = jnp.min(jnp.where(p2 == m2, iota, E), axis=1, keepdims=True)
    s = m1 + m2
    w_ref[...] = jnp.concatenate(
        [jnp.broadcast_to(m1 / s, (T, LANES)),
         jnp.broadcast_to(m2 / s, (T, LANES))], axis=1)

    oh1 = (iota == i1).astype(jnp.float32)
    oh2 = (iota == i2).astype(jnp.float32)
    cmat = oh1 + oh2
    r_i = jax.lax.broadcasted_iota(jnp.int32, (BS, BS), 0)
    c_i = jax.lax.broadcasted_iota(jnp.int32, (BS, BS), 1)
    lts = (c_i < r_i).astype(jnp.float32)
    counts = jnp.zeros((1, E), jnp.float32)
    blocks = []
    for b in range(NB):
        cb = cmat[b * BS:(b + 1) * BS, :]
        blocks.append(jnp.dot(lts, cb, preferred_element_type=jnp.float32)
                      + counts)
        counts = counts + jnp.sum(cb, axis=0, keepdims=True)
    cum = jnp.concatenate(blocks, axis=0)

    ue = (jax.lax.broadcasted_iota(jnp.int32, (E, E), 0)
          <= jax.lax.broadcasted_iota(jnp.int32, (E, E), 1)).astype(jnp.float32)
    grp_end = jnp.dot(counts, ue, preferred_element_type=jnp.float32)
    grp_start = grp_end - counts

    slot1 = jnp.sum(oh1 * (grp_start + cum), axis=1, keepdims=True)
    slot2 = jnp.sum(oh2 * (grp_start + cum), axis=1, keepdims=True)
    slots_ref[...] = jnp.concatenate([slot1, slot2], axis=1).astype(jnp.int32)
    sa_ref[...] = slot1.astype(jnp.int32)
    sb_ref[...] = slot2.astype(jnp.int32)

    first_t = jnp.floor(grp_start * (1.0 / TM))
    last_t = jnp.floor((grp_end - 1.0) * (1.0 / TM))
    ntiles = jnp.where(counts > 0.0, last_t - first_t + 1.0, 0.0)
    s_end = jnp.dot(ntiles, ue, preferred_element_type=jnp.float32)
    s_start = s_end - ntiles
    total_steps = jnp.sum(ntiles)

    srow = jax.lax.broadcasted_iota(jnp.int32, (NT_PAD, E), 0).astype(jnp.float32)
    s_clamp = jnp.minimum(srow, total_steps - 1.0)
    cmp = (jnp.broadcast_to(s_end, (NT_PAD, E)) <= s_clamp).astype(jnp.float32)
    se = jnp.minimum(jnp.sum(cmp, axis=1, keepdims=True), E - 1)
    oh_s = (jax.lax.broadcasted_iota(jnp.int32, (NT_PAD, E), 1)
            == se.astype(jnp.int32)).astype(jnp.float32)
    gs_s = jnp.sum(oh_s * grp_start, axis=1, keepdims=True)
    ge_s = jnp.sum(oh_s * grp_end, axis=1, keepdims=True)
    ft_s = jnp.sum(oh_s * first_t, axis=1, keepdims=True)
    ss_s = jnp.sum(oh_s * s_start, axis=1, keepdims=True)
    sid = jax.lax.broadcasted_iota(jnp.int32, (NT_PAD, 1), 0)
    sid_f = sid.astype(jnp.float32)
    tile = ft_s + (sid_f - ss_s)
    live = sid_f < total_steps
    tile_i = tile.astype(jnp.int32)
    prev_tile = jnp.concatenate([-jnp.ones((1, 1), jnp.float32), tile[:-1]],
                                axis=0)
    first = jnp.logical_and(live, tile != prev_tile)
    te_ref[...] = se.astype(jnp.int32)
    tx_ref[...] = jnp.where(live, tile_i, 0)
    ty_ref[...] = jnp.where(live, tile_i, NX)
    sf_ref[...] = first.astype(jnp.int32)
    st_ref[...] = tile_i
    gs_ref[...] = gs_s.astype(jnp.int32)
    ge_ref[...] = ge_s.astype(jnp.int32)


def _ffn_kernel(te_ref, tx_ref, ty_ref, sf_ref, st_ref, gs_ref, ge_ref,
                x_ref, wgu_ref, wd_ref, y_ref):
    i = pl.program_id(0)

    @pl.when(ty_ref[i] != NX)
    def _():
        gu = jnp.dot(x_ref[...], wgu_ref[0],
                     preferred_element_type=jnp.float32)
        g = gu[:, :F]
        u = gu[:, F:]
        h = g * jax.nn.sigmoid(g) * u
        y = jnp.dot(h, wd_ref[0], preferred_element_type=jnp.float32)
        base_row = st_ref[i] * TM
        rows = jax.lax.broadcasted_iota(jnp.int32, (TM, 1), 0)
        mask = jnp.logical_and(rows >= gs_ref[i] - base_row,
                               rows < ge_ref[i] - base_row)
        ym = y * mask.astype(jnp.float32)

        @pl.when(sf_ref[i] == 1)
        def _():
            y_ref[...] = ym

        @pl.when(sf_ref[i] == 0)
        def _():
            y_ref[...] = y_ref[...] + ym


HG = GB // 2


def _sc_dispatch_body(hs_hbm, tok_hbm, slot_hbm, xs_hbm,
                      tok0_v, tok1_v, slot0_v, slot1_v, rows0_v, rows1_v,
                      sem_i, sem_g0, sem_g1, sem_s):
    wid = lax.axis_index("s") * 2 + lax.axis_index("c")
    base = wid * GB
    c0 = pltpu.async_copy(tok_hbm.at[pl.ds(base, HG)], tok0_v, sem_i)
    c1 = pltpu.async_copy(tok_hbm.at[pl.ds(base + HG, HG)], tok1_v, sem_i)
    c2 = pltpu.async_copy(slot_hbm.at[pl.ds(base, HG)], slot0_v, sem_i)
    c3 = pltpu.async_copy(slot_hbm.at[pl.ds(base + HG, HG)], slot1_v, sem_i)
    c0.wait()
    c1.wait()
    c2.wait()
    c3.wait()
    g0 = pltpu.async_copy(hs_hbm.at[tok0_v], rows0_v, sem_g0)
    g1 = pltpu.async_copy(hs_hbm.at[tok1_v], rows1_v, sem_g1)
    g0.wait()
    s0 = pltpu.async_copy(rows0_v, xs_hbm.at[slot0_v], sem_s)
    g1.wait()
    s1 = pltpu.async_copy(rows1_v, xs_hbm.at[slot1_v], sem_s)
    s0.wait()
    s1.wait()


HC = CB // 2


def _sc_combine_body(ys_hbm, sa_hbm, sb_hbm, wab_hbm, out_hbm,
                     sa_v, sb_v, ya_v, yb_v, wab_v,
                     sem_i, sem_a, sem_b, sem_o):
    wid = lax.axis_index("s") * 2 + lax.axis_index("c")
    base = wid * CB
    c0 = pltpu.async_copy(sa_hbm.at[pl.ds(base, CB)], sa_v, sem_i)
    c1 = pltpu.async_copy(sb_hbm.at[pl.ds(base, CB)], sb_v, sem_i)
    c2 = pltpu.async_copy(wab_hbm.at[pl.ds(base, CB)], wab_v, sem_i)
    c0.wait()
    c1.wait()
    c2.wait()
    ga0 = pltpu.async_copy(ys_hbm.at[sa_v.at[pl.ds(0, HC)]],
                           ya_v.at[pl.ds(0, HC)], sem_a)
    gb0 = pltpu.async_copy(ys_hbm.at[sb_v.at[pl.ds(0, HC)]],
                           yb_v.at[pl.ds(0, HC)], sem_a)
    ga1 = pltpu.async_copy(ys_hbm.at[sa_v.at[pl.ds(HC, HC)]],
                           ya_v.at[pl.ds(HC, HC)], sem_b)
    gb1 = pltpu.async_copy(ys_hbm.at[sb_v.at[pl.ds(HC, HC)]],
                           yb_v.at[pl.ds(HC, HC)], sem_b)

    def row_body(r, carry):
        wa = wab_v[r, pl.ds(0, LANES)]
        wb = wab_v[r, pl.ds(LANES, LANES)]
        for c in range(DC):
            ya = ya_v[r, pl.ds(c * LANES, LANES)]
            yb = yb_v[r, pl.ds(c * LANES, LANES)]
            ya_v[r, pl.ds(c * LANES, LANES)] = wa * ya + wb * yb
        return carry

    ga0.wait()
    gb0.wait()
    lax.fori_loop(0, HC, row_body, 0)
    o0 = pltpu.async_copy(ya_v.at[pl.ds(0, HC)],
                          out_hbm.at[pl.ds(base, HC)], sem_o)
    ga1.wait()
    gb1.wait()
    lax.fori_loop(HC, CB, row_body, 0)
    o1 = pltpu.async_copy(ya_v.at[pl.ds(HC, HC)],
                          out_hbm.at[pl.ds(base + HC, HC)], sem_o)
    o0.wait()
    o1.wait()


def kernel(hidden_states, W_gate, W_gate_up, W_down, num_global_tokens,
           max_num_tokens_per_gpu):
    hs = hidden_states
    wab, slots, sa, sb, te2, tx2, ty2, sf2, st2, gs2, ge2 = pl.pallas_call(
        _router_kernel,
        out_shape=(
            jax.ShapeDtypeStruct((T, 2 * LANES), jnp.float32),
            jax.ShapeDtypeStruct((T, K), jnp.int32),
            jax.ShapeDtypeStruct((T, 1), jnp.int32),
            jax.ShapeDtypeStruct((T, 1), jnp.int32),
            jax.ShapeDtypeStruct((NT_PAD, 1), jnp.int32),
            jax.ShapeDtypeStruct((NT_PAD, 1), jnp.int32),
            jax.ShapeDtypeStruct((NT_PAD, 1), jnp.int32),
            jax.ShapeDtypeStruct((NT_PAD, 1), jnp.int32),
            jax.ShapeDtypeStruct((NT_PAD, 1), jnp.int32),
            jax.ShapeDtypeStruct((NT_PAD, 1), jnp.int32),
            jax.ShapeDtypeStruct((NT_PAD, 1), jnp.int32),
        ),
    )(hs, W_gate)
    tile_expert = te2.reshape(-1)
    tile_x = tx2.reshape(-1)
    tile_y = ty2.reshape(-1)
    tile_f = sf2.reshape(-1)
    tile_t = st2.reshape(-1)
    tile_gs = gs2.reshape(-1)
    tile_ge = ge2.reshape(-1)
    t_flat = jnp.repeat(jnp.arange(T, dtype=jnp.int32), K)
    slot = slots.reshape(-1)

    sc_mesh = plsc.VectorSubcoreMesh(core_axis_name="c", subcore_axis_name="s")
    sc_dispatch = pl.kernel(
        _sc_dispatch_body,
        mesh=sc_mesh,
        out_type=jax.ShapeDtypeStruct((NG, D), jnp.float32),
        scratch_types=[
            pltpu.VMEM((HG,), jnp.int32),
            pltpu.VMEM((HG,), jnp.int32),
            pltpu.VMEM((HG,), jnp.int32),
            pltpu.VMEM((HG,), jnp.int32),
            pltpu.VMEM((HG, D), jnp.float32),
            pltpu.VMEM((HG, D), jnp.float32),
            pltpu.SemaphoreType.DMA,
            pltpu.SemaphoreType.DMA,
            pltpu.SemaphoreType.DMA,
            pltpu.SemaphoreType.DMA,
        ],
    )
    x_sorted = sc_dispatch(hs, t_flat, slot)

    grid_spec = pltpu.PrefetchScalarGridSpec(
        num_scalar_prefetch=7,
        grid=(N_TILES,),
        in_specs=[
            pl.BlockSpec((TM, D),
                         lambda i, te, tx, ty, sf, st, gs, ge: (tx[i], 0)),
            pl.BlockSpec((1, D, 2 * F),
                         lambda i, te, tx, ty, sf, st, gs, ge: (te[i], 0, 0)),
            pl.BlockSpec((1, F, D),
                         lambda i, te, tx, ty, sf, st, gs, ge: (te[i], 0, 0)),
        ],
        out_specs=pl.BlockSpec((TM, D),
                               lambda i, te, tx, ty, sf, st, gs, ge: (ty[i], 0)),
    )
    y_sorted = pl.pallas_call(
        _ffn_kernel,
        grid_spec=grid_spec,
        out_shape=jax.ShapeDtypeStruct(((NX + 1) * TM, D), jnp.float32),
    )(tile_expert, tile_x, tile_y, tile_f, tile_t, tile_gs, tile_ge,
      x_sorted, W_gate_up, W_down)

    sc_combine = pl.kernel(
        _sc_combine_body,
        mesh=sc_mesh,
        out_type=jax.ShapeDtypeStruct((T, D), jnp.float32),
        scratch_types=[
            pltpu.VMEM((CB,), jnp.int32),
            pltpu.VMEM((CB,), jnp.int32),
            pltpu.VMEM((CB, D), jnp.float32),
            pltpu.VMEM((CB, D), jnp.float32),
            pltpu.VMEM((CB, 2 * LANES), jnp.float32),
            pltpu.SemaphoreType.DMA,
            pltpu.SemaphoreType.DMA,
            pltpu.SemaphoreType.DMA,
            pltpu.SemaphoreType.DMA,
        ],
    )
    out = sc_combine(y_sorted, sa.reshape(T), sb.reshape(T), wab)
    return out


# speedup vs baseline: 1.2147x; 1.2147x over previous
import jax
import jax.numpy as jnp
from jax import lax
from jax.experimental import pallas as pl
from jax.experimental.pallas import tpu as pltpu
from jax.experimental.pallas import tpu_sc as plsc

T = 2048
D = 768
E = 64
K = 2
F = 512

TM = 128
N_TILES = (T * K) // TM + (E - 1)
NG = N_TILES * TM

NW = 32
GB = (T * K) // NW
CB = T // NW
LANES = 16
DC = D // LANES


BS = 256
NB = T // BS
NT_PAD = 128


def _router_kernel(hs_ref, wg_ref, w_ref, slots_ref, sa_ref, sb_ref,
                   te_ref, tx_ref, ty_ref):
    logits = jnp.dot(hs_ref[...], wg_ref[...], preferred_element_type=jnp.float32)
    m = jnp.max(logits, axis=1, keepdims=True)
    p = jnp.exp(logits - m)
    p = p / jnp.sum(p, axis=1, keepdims=True)
    iota = jax.lax.broadcasted_iota(jnp.int32, (T, E), 1)
    m1 = jnp.max(p, axis=1, keepdims=True)
    i1 = jnp.min(jnp.where(p == m1, iota, E), axis=1, keepdims=True)
    p2 = jnp.where(iota == i1, -1e30, p)
    m2 = jnp.max(p2, axis=1, keepdims=True)
    i2 = jnp.min(jnp.where(p2 == m2, iota, E), axis=1, keepdims=True)
    s = m1 + m2
    w_ref[...] = jnp.concatenate(
        [jnp.broadcast_to(m1 / s, (T, LANES)),
         jnp.broadcast_to(m2 / s, (T, LANES))], axis=1)

    oh1 = (iota == i1).astype(jnp.float32)
    oh2 = (iota == i2).astype(jnp.float32)
    cmat = oh1 + oh2
    r_i = jax.lax.broadcasted_iota(jnp.int32, (BS, BS), 0)
    c_i = jax.lax.broadcasted_iota(jnp.int32, (BS, BS), 1)
    lts = (c_i < r_i).astype(jnp.float32)
    counts = jnp.zeros((1, E), jnp.float32)
    blocks = []
    for b in range(NB):
        cb = cmat[b * BS:(b + 1) * BS, :]
        blocks.append(jnp.dot(lts, cb, preferred_element_type=jnp.float32)
                      + counts)
        counts = counts + jnp.sum(cb, axis=0, keepdims=True)
    cum = jnp.concatenate(blocks, axis=0)

    padded = jnp.floor((counts + (TM - 1)) * (1.0 / TM)) * TM
    ue = (jax.lax.broadcasted_iota(jnp.int32, (E, E), 0)
          <= jax.lax.broadcasted_iota(jnp.int32, (E, E), 1)).astype(jnp.float32)
    pad_end = jnp.dot(padded, ue, preferred_element_type=jnp.float32)
    pad_start = pad_end - padded
    total = jnp.sum(padded)

    slot1 = jnp.sum(oh1 * (pad_start + cum), axis=1, keepdims=True)
    slot2 = jnp.sum(oh2 * (pad_start + cum), axis=1, keepdims=True)
    slots_ref[...] = jnp.concatenate([slot1, slot2], axis=1).astype(jnp.int32)
    sa_ref[...] = slot1.astype(jnp.int32)
    sb_ref[...] = slot2.astype(jnp.int32)

    tcol = jax.lax.broadcasted_iota(jnp.int32, (NT_PAD, E), 0) * TM
    tscol = jnp.minimum(tcol.astype(jnp.float32), total - 1.0)
    cmp = (jnp.broadcast_to(pad_end, (NT_PAD, E)) <= tscol).astype(jnp.float32)
    te = jnp.minimum(jnp.sum(cmp, axis=1, keepdims=True), E - 1)
    te_ref[...] = te.astype(jnp.int32)
    live = tcol[:, :1].astype(jnp.float32) < total
    tid = jax.lax.broadcasted_iota(jnp.int32, (NT_PAD, 1), 0)
    tx_ref[...] = jnp.where(live, tid, 0)
    ty_ref[...] = jnp.where(live, tid, N_TILES)


def _ffn_kernel(te_ref, tx_ref, ty_ref, x_ref, wgu_ref, wd_ref, y_ref):
    i = pl.program_id(0)

    @pl.when(ty_ref[i] != N_TILES)
    def _():
        gu = jnp.dot(x_ref[...], wgu_ref[0],
                     preferred_element_type=jnp.float32)
        g = gu[:, :F]
        u = gu[:, F:]
        h = g * jax.nn.sigmoid(g) * u
        y_ref[...] = jnp.dot(h, wd_ref[0], preferred_element_type=jnp.float32)


HG = GB // 2


def _sc_dispatch_body(hs_hbm, tok_hbm, slot_hbm, xs_hbm,
                      tok0_v, tok1_v, slot0_v, slot1_v, rows0_v, rows1_v,
                      sem_i, sem_g0, sem_g1, sem_s):
    wid = lax.axis_index("s") * 2 + lax.axis_index("c")
    base = wid * GB
    c0 = pltpu.async_copy(tok_hbm.at[pl.ds(base, HG)], tok0_v, sem_i)
    c1 = pltpu.async_copy(tok_hbm.at[pl.ds(base + HG, HG)], tok1_v, sem_i)
    c2 = pltpu.async_copy(slot_hbm.at[pl.ds(base, HG)], slot0_v, sem_i)
    c3 = pltpu.async_copy(slot_hbm.at[pl.ds(base + HG, HG)], slot1_v, sem_i)
    c0.wait()
    c1.wait()
    c2.wait()
    c3.wait()
    g0 = pltpu.async_copy(hs_hbm.at[tok0_v], rows0_v, sem_g0)
    g1 = pltpu.async_copy(hs_hbm.at[tok1_v], rows1_v, sem_g1)
    g0.wait()
    s0 = pltpu.async_copy(rows0_v, xs_hbm.at[slot0_v], sem_s)
    g1.wait()
    s1 = pltpu.async_copy(rows1_v, xs_hbm.at[slot1_v], sem_s)
    s0.wait()
    s1.wait()


HC = CB // 2


def _sc_combine_body(ys_hbm, sa_hbm, sb_hbm, wab_hbm, out_hbm,
                     sa_v, sb_v, ya_v, yb_v, wab_v,
                     sem_i, sem_a, sem_b, sem_o):
    wid = lax.axis_index("s") * 2 + lax.axis_index("c")
    base = wid * CB
    c0 = pltpu.async_copy(sa_hbm.at[pl.ds(base, CB)], sa_v, sem_i)
    c1 = pltpu.async_copy(sb_hbm.at[pl.ds(base, CB)], sb_v, sem_i)
    c2 = pltpu.async_copy(wab_hbm.at[pl.ds(base, CB)], wab_v, sem_i)
    c0.wait()
    c1.wait()
    c2.wait()
    ga0 = pltpu.async_copy(ys_hbm.at[sa_v.at[pl.ds(0, HC)]],
                           ya_v.at[pl.ds(0, HC)], sem_a)
    gb0 = pltpu.async_copy(ys_hbm.at[sb_v.at[pl.ds(0, HC)]],
                           yb_v.at[pl.ds(0, HC)], sem_a)
    ga1 = pltpu.async_copy(ys_hbm.at[sa_v.at[pl.ds(HC, HC)]],
                           ya_v.at[pl.ds(HC, HC)], sem_b)
    gb1 = pltpu.async_copy(ys_hbm.at[sb_v.at[pl.ds(HC, HC)]],
                           yb_v.at[pl.ds(HC, HC)], sem_b)

    def row_body(r, carry):
        wa = wab_v[r, pl.ds(0, LANES)]
        wb = wab_v[r, pl.ds(LANES, LANES)]
        for c in range(DC):
            ya = ya_v[r, pl.ds(c * LANES, LANES)]
            yb = yb_v[r, pl.ds(c * LANES, LANES)]
            ya_v[r, pl.ds(c * LANES, LANES)] = wa * ya + wb * yb
        return carry

    ga0.wait()
    gb0.wait()
    lax.fori_loop(0, HC, row_body, 0)
    o0 = pltpu.async_copy(ya_v.at[pl.ds(0, HC)],
                          out_hbm.at[pl.ds(base, HC)], sem_o)
    ga1.wait()
    gb1.wait()
    lax.fori_loop(HC, CB, row_body, 0)
    o1 = pltpu.async_copy(ya_v.at[pl.ds(HC, HC)],
                          out_hbm.at[pl.ds(base + HC, HC)], sem_o)
    o0.wait()
    o1.wait()


def kernel(hidden_states, W_gate, W_gate_up, W_down, num_global_tokens,
           max_num_tokens_per_gpu):
    hs = hidden_states
    wab, slots, sa, sb, te2, tx2, ty2 = pl.pallas_call(
        _router_kernel,
        out_shape=(
            jax.ShapeDtypeStruct((T, 2 * LANES), jnp.float32),
            jax.ShapeDtypeStruct((T, K), jnp.int32),
            jax.ShapeDtypeStruct((T, 1), jnp.int32),
            jax.ShapeDtypeStruct((T, 1), jnp.int32),
            jax.ShapeDtypeStruct((NT_PAD, 1), jnp.int32),
            jax.ShapeDtypeStruct((NT_PAD, 1), jnp.int32),
            jax.ShapeDtypeStruct((NT_PAD, 1), jnp.int32),
        ),
    )(hs, W_gate)
    tile_expert = te2.reshape(-1)
    tile_x = tx2.reshape(-1)
    tile_y = ty2.reshape(-1)
    t_flat = jnp.repeat(jnp.arange(T, dtype=jnp.int32), K)
    slot = slots.reshape(-1)

    sc_mesh = plsc.VectorSubcoreMesh(core_axis_name="c", subcore_axis_name="s")
    sc_dispatch = pl.kernel(
        _sc_dispatch_body,
        mesh=sc_mesh,
        out_type=jax.ShapeDtypeStruct((NG, D), jnp.float32),
        scratch_types=[
            pltpu.VMEM((HG,), jnp.int32),
            pltpu.VMEM((HG,), jnp.int32),
            pltpu.VMEM((HG,), jnp.int32),
            pltpu.VMEM((HG,), jnp.int32),
            pltpu.VMEM((HG, D), jnp.float32),
            pltpu.VMEM((HG, D), jnp.float32),
            pltpu.SemaphoreType.DMA,
            pltpu.SemaphoreType.DMA,
            pltpu.SemaphoreType.DMA,
            pltpu.SemaphoreType.DMA,
        ],
    )
    x_sorted = sc_dispatch(hs, t_flat, slot)

    grid_spec = pltpu.PrefetchScalarGridSpec(
        num_scalar_prefetch=3,
        grid=(N_TILES,),
        in_specs=[
            pl.BlockSpec((TM, D), lambda i, te, tx, ty: (tx[i], 0)),
            pl.BlockSpec((1, D, 2 * F), lambda i, te, tx, ty: (te[i], 0, 0)),
            pl.BlockSpec((1, F, D), lambda i, te, tx, ty: (te[i], 0, 0)),
        ],
        out_specs=pl.BlockSpec((TM, D), lambda i, te, tx, ty: (ty[i], 0)),
    )
    y_sorted = pl.pallas_call(
        _ffn_kernel,
        grid_spec=grid_spec,
        out_shape=jax.ShapeDtypeStruct(((N_TILES + 1) * TM, D), jnp.float32),
    )(tile_expert, tile_x, tile_y, x_sorted, W_gate_up, W_down)

    sc_combine = pl.kernel(
        _sc_combine_body,
        mesh=sc_mesh,
        out_type=jax.ShapeDtypeStruct((T, D), jnp.float32),
        scratch_types=[
            pltpu.VMEM((CB,), jnp.int32),
            pltpu.VMEM((CB,), jnp.int32),
            pltpu.VMEM((CB, D), jnp.float32),
            pltpu.VMEM((CB, D), jnp.float32),
            pltpu.VMEM((CB, 2 * LANES), jnp.float32),
            pltpu.SemaphoreType.DMA,
            pltpu.SemaphoreType.DMA,
            pltpu.SemaphoreType.DMA,
            pltpu.SemaphoreType.DMA,
        ],
    )
    out = sc_combine(y_sorted, sa.reshape(T), sb.reshape(T), wab)
    return out
